# baseline (speedup 1.0000x reference)
import jax
import jax.numpy as jnp
from jax.experimental import pallas as pl

_TEMP = 2.0
_K = 32
_ROWS = 256


def _latents_kernel(x_ref, out_ref):
    x = x_ref[...]
    r, dp = x.shape
    inv_t = jnp.float32(1.0 / _TEMP)
    m = jnp.max(x, axis=1, keepdims=True)
    s = jnp.sum(jnp.exp((x - m) * inv_t), axis=1, keepdims=True)
    col = jax.lax.broadcasted_iota(jnp.int32, (r, dp), 1)
    neginf = jnp.float32(-jnp.inf)

    def body(_, carry):
        xc, out, denom = carry
        m1 = jnp.max(xc, axis=1, keepdims=True)
        idx = jnp.argmax(xc, axis=1).astype(jnp.int32)[:, None]
        sel = col == idx
        ev = jnp.exp((m1 - m) * inv_t)
        out = jnp.where(sel, ev / denom, out)
        xc = jnp.where(sel, neginf, xc)
        return xc, out, denom - ev

    _, out, _ = jax.lax.fori_loop(
        0, _K, body, (x, jnp.zeros_like(x), s), unroll=True
    )
    out_ref[...] = out


def kernel(normu, cls):
    n, d = cls.shape
    dp = 1024
    xp = jnp.pad(cls, ((0, 0), (0, dp - d)), constant_values=-jnp.inf)
    out = pl.pallas_call(
        _latents_kernel,
        grid=(n // _ROWS,),
        in_specs=[pl.BlockSpec((_ROWS, dp), lambda i: (i, 0))],
        out_specs=pl.BlockSpec((_ROWS, dp), lambda i: (i, 0)),
        out_shape=jax.ShapeDtypeStruct((n, dp), jnp.float32),
    )(xp)
    return (normu, out[:, :d])


# speedup vs baseline: 2.3438x; 2.3438x over previous
import jax
import jax.numpy as jnp
from jax.experimental import pallas as pl

_TEMP = 2.0
_K = 32
_ROWS = 512


def _latents_kernel(x_ref, out_ref):
    x = x_ref[...]
    r, dp = x.shape
    inv_t = jnp.float32(1.0 / _TEMP)
    m = jnp.max(x, axis=1, keepdims=True)
    s = jnp.sum(jnp.exp((x - m) * inv_t), axis=1, keepdims=True)
    col = jax.lax.broadcasted_iota(jnp.int32, (r, dp), 1)
    neginf = jnp.float32(-jnp.inf)

    def body(_, carry):
        xc, out, denom = carry
        m1 = jnp.max(xc, axis=1, keepdims=True)
        idx = jnp.min(jnp.where(xc == m1, col, dp), axis=1, keepdims=True)
        sel = col == idx
        ev = jnp.exp((m1 - m) * inv_t)
        out = jnp.where(sel, ev / denom, out)
        xc = jnp.where(sel, neginf, xc)
        return xc, out, denom - ev

    _, out, _ = jax.lax.fori_loop(
        0, _K, body, (x, jnp.zeros_like(x), s), unroll=True
    )
    out_ref[...] = out


def kernel(normu, cls):
    n, d = cls.shape
    dp = 1024
    xp = jnp.pad(cls, ((0, 0), (0, dp - d)), constant_values=-jnp.inf)
    out = pl.pallas_call(
        _latents_kernel,
        grid=(n // _ROWS,),
        in_specs=[pl.BlockSpec((_ROWS, dp), lambda i: (i, 0))],
        out_specs=pl.BlockSpec((_ROWS, dp), lambda i: (i, 0)),
        out_shape=jax.ShapeDtypeStruct((n, dp), jnp.float32),
    )(xp)
    return (normu, out[:, :d])
